# baseline (speedup 1.0000x reference)
import functools

import jax
import jax.numpy as jnp
from jax import lax
from jax.experimental import pallas as pl
from jax.experimental.pallas import tpu as pltpu
from jax.experimental.pallas import tpu_sc as plsc

E = 8
H = 1024
FFN = 2816
T = 4096
A = 2 * T
TM = 512
P = A + E * TM
NT = P // TM
TF = 1408
NF = FFN // TF
NEG = -1e30

NW = 32
TPW = T // NW
H32 = H // 2

TMR = 512


def _router_body(lg_ref, x_ref, i1_ref, i2_ref, w1_ref, w2_ref, r0_ref, r1_ref,
                 cnt_ref, xb_ref, carry):
    step = pl.program_id(0)

    @pl.when(step == 0)
    def _():
        carry[...] = jnp.zeros_like(carry)

    xb_ref[...] = x_ref[...].astype(jnp.bfloat16)
    logits = lg_ref[...]
    iota = lax.broadcasted_iota(jnp.int32, logits.shape, 1)
    l1 = jnp.max(logits, axis=1, keepdims=True)
    i1 = jnp.min(jnp.where(logits == l1, iota, E), axis=1, keepdims=True)
    m2 = jnp.where(iota == i1, NEG, logits)
    l2 = jnp.max(m2, axis=1, keepdims=True)
    i2 = jnp.min(jnp.where(m2 == l2, iota, E), axis=1, keepdims=True)
    w1 = 1.0 / (1.0 + jnp.exp(l2 - l1))
    i1_ref[...] = i1
    i2_ref[...] = i2
    w1_ref[...] = w1
    w2_ref[...] = 1.0 - w1

    oh1 = (iota == i1).astype(jnp.float32)
    oh2 = (iota == i2).astype(jnp.float32)
    both = oh1 + oh2
    rr = lax.broadcasted_iota(jnp.int32, (TMR, TMR), 0)
    cc = lax.broadcasted_iota(jnp.int32, (TMR, TMR), 1)
    tri = (cc < rr).astype(jnp.float32)
    ex = lax.dot_general(tri, both, (((1,), (0,)), ((), ())),
                         precision=jax.lax.Precision.HIGHEST,
                         preferred_element_type=jnp.float32) + carry[...]
    r0_ref[...] = jnp.sum(ex * oh1, axis=1, keepdims=True).astype(jnp.int32)
    r1_ref[...] = jnp.sum(ex * oh2, axis=1, keepdims=True).astype(jnp.int32)
    carry[...] += jnp.sum(both, axis=0, keepdims=True)
    cnt_ref[...] = carry[...]


def _router(logits, x):
    return pl.pallas_call(
        _router_body,
        grid=(T // TMR,),
        in_specs=[
            pl.BlockSpec((TMR, E), lambda i: (i, 0)),
            pl.BlockSpec((TMR, H), lambda i: (i, 0)),
        ],
        out_specs=[
            pl.BlockSpec((TMR, 1), lambda i: (i, 0)),
            pl.BlockSpec((TMR, 1), lambda i: (i, 0)),
            pl.BlockSpec((TMR, 1), lambda i: (i, 0)),
            pl.BlockSpec((TMR, 1), lambda i: (i, 0)),
            pl.BlockSpec((TMR, 1), lambda i: (i, 0)),
            pl.BlockSpec((TMR, 1), lambda i: (i, 0)),
            pl.BlockSpec((1, E), lambda i: (0, 0)),
            pl.BlockSpec((TMR, H), lambda i: (i, 0)),
        ],
        out_shape=[
            jax.ShapeDtypeStruct((T, 1), jnp.int32),
            jax.ShapeDtypeStruct((T, 1), jnp.int32),
            jax.ShapeDtypeStruct((T, 1), jnp.float32),
            jax.ShapeDtypeStruct((T, 1), jnp.float32),
            jax.ShapeDtypeStruct((T, 1), jnp.int32),
            jax.ShapeDtypeStruct((T, 1), jnp.int32),
            jax.ShapeDtypeStruct((1, E), jnp.float32),
            jax.ShapeDtypeStruct((T, H), jnp.bfloat16),
        ],
        scratch_shapes=[pltpu.VMEM((1, E), jnp.float32)],
    )(logits, x)


def _meta_body(cnt_ref, i1_ref, i2_ref, r0_ref, r1_ref,
               d0_ref, d1_ref, te_ref, act_ref):
    cnt = cnt_ref[...]
    padded = jnp.ceil(cnt * (1.0 / TM)) * TM
    er = lax.broadcasted_iota(jnp.int32, (E, E), 0)
    ec = lax.broadcasted_iota(jnp.int32, (E, E), 1)
    lt = (er < ec).astype(jnp.float32)
    starts = lax.dot_general(padded, lt, (((1,), (0,)), ((), ())),
                             precision=jax.lax.Precision.HIGHEST,
                             preferred_element_type=jnp.float32)
    ends = starts + padded
    iota1 = lax.broadcasted_iota(jnp.int32, (T, E), 1)
    s1 = jnp.sum(jnp.where(iota1 == i1_ref[...], starts, 0.0), axis=1,
                 keepdims=True)
    s2 = jnp.sum(jnp.where(iota1 == i2_ref[...], starts, 0.0), axis=1,
                 keepdims=True)
    d0_ref[...] = (s1 + r0_ref[...].astype(jnp.float32)).astype(jnp.int32)
    d1_ref[...] = (s2 + r1_ref[...].astype(jnp.float32)).astype(jnp.int32)

    tb = (lax.broadcasted_iota(jnp.int32, (NT, 1), 0) * TM).astype(jnp.float32)
    te_raw = jnp.minimum(
        jnp.sum((tb >= ends).astype(jnp.int32), axis=1, keepdims=True), E - 1)
    total = jnp.max(ends, axis=1, keepdims=True)
    act = (tb < total).astype(jnp.int32)
    nact = jnp.sum(act, axis=0, keepdims=True)
    ti = lax.broadcasted_iota(jnp.int32, (NT, 1), 0)
    last_e = jnp.sum(jnp.where(ti == nact - 1, te_raw, 0), axis=0,
                     keepdims=True)
    te_ref[...] = jnp.where(act == 1, te_raw, last_e)
    act_ref[...] = act


def _meta(cnt, i1, i2, r0, r1):
    return pl.pallas_call(
        _meta_body,
        out_shape=[
            jax.ShapeDtypeStruct((T, 1), jnp.int32),
            jax.ShapeDtypeStruct((T, 1), jnp.int32),
            jax.ShapeDtypeStruct((NT, 1), jnp.int32),
            jax.ShapeDtypeStruct((NT, 1), jnp.int32),
        ],
    )(cnt, i1, i2, r0, r1)


@functools.cache
def _sc_dispatch_fn():
    mesh = plsc.VectorSubcoreMesh(core_axis_name="c", subcore_axis_name="s")

    @functools.partial(
        pl.kernel,
        mesh=mesh,
        out_type=jax.ShapeDtypeStruct((P, H32), jnp.int32),
        scratch_types=[
            pltpu.VMEM((2, TPW), jnp.int32),
            pltpu.VMEM((TPW, H32), jnp.int32),
        ],
    )
    def _sc_dispatch(x_hbm, dst_hbm, xg_hbm, dst_v, rows_v):
        wid = lax.axis_index("s") * 2 + lax.axis_index("c")
        pltpu.sync_copy(dst_hbm.at[wid], dst_v)
        base = wid * TPW
        pltpu.sync_copy(x_hbm.at[pl.ds(base, TPW)], rows_v)
        pltpu.sync_copy(rows_v, xg_hbm.at[dst_v.at[0]])
        pltpu.sync_copy(rows_v, xg_hbm.at[dst_v.at[1]])

    return _sc_dispatch


@functools.cache
def _sc_gather2_fn():
    mesh = plsc.VectorSubcoreMesh(core_axis_name="c", subcore_axis_name="s")

    @functools.partial(
        pl.kernel,
        mesh=mesh,
        out_type=[jax.ShapeDtypeStruct((T, H32), jnp.int32),
                  jax.ShapeDtypeStruct((T, H32), jnp.int32)],
        scratch_types=[
            pltpu.VMEM((2, TPW), jnp.int32),
            pltpu.VMEM((TPW, H32), jnp.int32),
        ],
    )
    def _sc_gather2(y_hbm, idx0_hbm, idx1_hbm, y0_hbm, y1_hbm, idx_v, rows_v):
        wid = lax.axis_index("s") * 2 + lax.axis_index("c")
        base = wid * TPW
        pltpu.sync_copy(idx0_hbm.at[pl.ds(base, TPW)], idx_v.at[0])
        pltpu.sync_copy(idx1_hbm.at[pl.ds(base, TPW)], idx_v.at[1])
        pltpu.sync_copy(y_hbm.at[idx_v.at[0]], rows_v)
        pltpu.sync_copy(rows_v, y0_hbm.at[pl.ds(base, TPW)])
        pltpu.sync_copy(y_hbm.at[idx_v.at[1]], rows_v)
        pltpu.sync_copy(rows_v, y1_hbm.at[pl.ds(base, TPW)])

    return _sc_gather2


def _mlp1_body(te_ref, act_ref, xg_ref, wg_ref, wu_ref, h_ref):
    @pl.when(act_ref[pl.program_id(1)] == 1)
    def _():
        x = xg_ref[...]
        wg = wg_ref[0].astype(jnp.bfloat16)
        wu = wu_ref[0].astype(jnp.bfloat16)
        g = jax.lax.dot_general(x, wg, (((1,), (1,)), ((), ())),
                                preferred_element_type=jnp.float32)
        u = jax.lax.dot_general(x, wu, (((1,), (1,)), ((), ())),
                                preferred_element_type=jnp.float32)
        h_ref[...] = ((g * (1.0 / (1.0 + jnp.exp(-g)))) * u).astype(jnp.bfloat16)


def _mlp1(te, act, xg, Wg, Wu):
    grid_spec = pltpu.PrefetchScalarGridSpec(
        num_scalar_prefetch=2,
        grid=(NF, NT),
        in_specs=[
            pl.BlockSpec((TM, H), lambda j, i, te, act: (i, 0)),
            pl.BlockSpec((1, TF, H), lambda j, i, te, act: (te[i], j, 0)),
            pl.BlockSpec((1, TF, H), lambda j, i, te, act: (te[i], j, 0)),
        ],
        out_specs=pl.BlockSpec((TM, TF), lambda j, i, te, act: (i, j)),
    )
    return pl.pallas_call(
        _mlp1_body,
        grid_spec=grid_spec,
        out_shape=jax.ShapeDtypeStruct((P, FFN), jnp.bfloat16),
    )(te, act, xg, Wg, Wu)


def _mlp2_body(te_ref, act_ref, h_ref, wd_ref, bd_ref, y_ref, acc):
    @pl.when(act_ref[pl.program_id(0)] == 1)
    def _():
        k = pl.program_id(1)

        @pl.when(k == 0)
        def _():
            acc[...] = jnp.broadcast_to(bd_ref[0], acc.shape)

        h = h_ref[...]
        wd = wd_ref[0].astype(jnp.bfloat16)
        acc[...] += jax.lax.dot_general(h, wd, (((1,), (1,)), ((), ())),
                                        preferred_element_type=jnp.float32)

        @pl.when(k == NF - 1)
        def _():
            y_ref[...] = acc[...].astype(jnp.bfloat16)


def _mlp2(te, act, h, Wd, bd):
    grid_spec = pltpu.PrefetchScalarGridSpec(
        num_scalar_prefetch=2,
        grid=(NT, NF),
        in_specs=[
            pl.BlockSpec((TM, TF), lambda i, k, te, act: (i, k)),
            pl.BlockSpec((1, H, TF), lambda i, k, te, act: (te[i], 0, k)),
            pl.BlockSpec((1, 1, H), lambda i, k, te, act: (te[i], 0, 0)),
        ],
        out_specs=pl.BlockSpec((TM, H), lambda i, k, te, act: (i, 0)),
        scratch_shapes=[pltpu.VMEM((TM, H), jnp.float32)],
    )
    return pl.pallas_call(
        _mlp2_body,
        grid_spec=grid_spec,
        out_shape=jax.ShapeDtypeStruct((P, H), jnp.bfloat16),
    )(te, act, h, Wd, bd.reshape(E, 1, H))


def _combine_body(res_ref, y1_ref, y2_ref, w1_ref, w2_ref, out_ref):
    out_ref[...] = (res_ref[...]
                    + w1_ref[...] * y1_ref[...].astype(jnp.float32)
                    + w2_ref[...] * y2_ref[...].astype(jnp.float32))


def _combine(res, y1, y2, w1, w2):
    return pl.pallas_call(
        _combine_body,
        grid=(T // TMR,),
        in_specs=[
            pl.BlockSpec((TMR, H), lambda i: (i, 0)),
            pl.BlockSpec((TMR, H), lambda i: (i, 0)),
            pl.BlockSpec((TMR, H), lambda i: (i, 0)),
            pl.BlockSpec((TMR, 1), lambda i: (i, 0)),
            pl.BlockSpec((TMR, 1), lambda i: (i, 0)),
        ],
        out_specs=pl.BlockSpec((TMR, H), lambda i: (i, 0)),
        out_shape=jax.ShapeDtypeStruct((T, H), jnp.float32),
    )(res, y1, y2, w1, w2)


def kernel(hidden_states, residual, expert_limit, Wr, Wg, Wu, Wd, bd):
    b, s, h = hidden_states.shape
    x = hidden_states.reshape(T, H)

    router_logits = jnp.einsum("bsh,eh->bse", hidden_states, Wr)
    expert_mask = jnp.arange(E) < expert_limit
    router_logits = jnp.where(expert_mask, router_logits, -jnp.inf)
    logits = router_logits.reshape(-1, E)
    logits_f = jnp.where(jnp.isfinite(logits), logits, NEG)
    i1, i2, w1, w2, r0, r1, cnt, xb = _router(logits_f, x)
    d0, d1, te2, act2 = _meta(cnt, i1, i2, r0, r1)
    te = te2.reshape(NT)
    act = act2.reshape(NT)

    dst_sw = jnp.concatenate(
        [d0.reshape(NW, 1, TPW), d1.reshape(NW, 1, TPW)], axis=1)
    xb32 = lax.bitcast_convert_type(xb.reshape(T, H32, 2), jnp.int32)
    xg32 = _sc_dispatch_fn()(xb32, dst_sw)
    xg = lax.bitcast_convert_type(xg32, jnp.bfloat16).reshape(P, H)

    yg = _mlp2(te, act, _mlp1(te, act, xg, Wg, Wu), Wd, bd)

    yg32 = lax.bitcast_convert_type(yg.reshape(P, H32, 2), jnp.int32)
    y1_32, y2_32 = _sc_gather2_fn()(yg32, d0.reshape(T), d1.reshape(T))
    y1 = lax.bitcast_convert_type(y1_32, jnp.bfloat16).reshape(T, H)
    y2 = lax.bitcast_convert_type(y2_32, jnp.bfloat16).reshape(T, H)

    out = _combine(residual.reshape(T, H), y1, y2, w1, w2)
    return out.reshape(b, s, h), logits


# speedup vs baseline: 2.6090x; 2.6090x over previous
import functools

import jax
import jax.numpy as jnp
from jax import lax
from jax.experimental import pallas as pl
from jax.experimental.pallas import tpu as pltpu
from jax.experimental.pallas import tpu_sc as plsc

E = 8
H = 1024
FFN = 2816
T = 4096
A = 2 * T
TM = 512
P = A + E * TM
NT = P // TM
TF = 1408
NF = FFN // TF
NEG = -1e30

NW = 32
TPW = T // NW
CH = 64
NCHD = TPW // CH

TMR = 512


def _router_body(lg_ref, i1_ref, i2_ref, w1_ref, w2_ref, r0_ref, r1_ref,
                 cnt_ref, carry):
    step = pl.program_id(0)

    @pl.when(step == 0)
    def _():
        carry[...] = jnp.zeros_like(carry)

    logits = lg_ref[...]
    iota = lax.broadcasted_iota(jnp.int32, logits.shape, 1)
    l1 = jnp.max(logits, axis=1, keepdims=True)
    i1 = jnp.min(jnp.where(logits == l1, iota, E), axis=1, keepdims=True)
    m2 = jnp.where(iota == i1, NEG, logits)
    l2 = jnp.max(m2, axis=1, keepdims=True)
    i2 = jnp.min(jnp.where(m2 == l2, iota, E), axis=1, keepdims=True)
    w1 = 1.0 / (1.0 + jnp.exp(l2 - l1))
    i1_ref[...] = i1
    i2_ref[...] = i2
    w1_ref[...] = w1
    w2_ref[...] = 1.0 - w1

    oh1 = (iota == i1).astype(jnp.float32)
    oh2 = (iota == i2).astype(jnp.float32)
    both = oh1 + oh2
    rr = lax.broadcasted_iota(jnp.int32, (TMR, TMR), 0)
    cc = lax.broadcasted_iota(jnp.int32, (TMR, TMR), 1)
    tri = (cc < rr).astype(jnp.float32)
    ex = lax.dot_general(tri, both, (((1,), (0,)), ((), ())),
                         precision=jax.lax.Precision.HIGHEST,
                         preferred_element_type=jnp.float32) + carry[...]
    r0_ref[...] = jnp.sum(ex * oh1, axis=1, keepdims=True).astype(jnp.int32)
    r1_ref[...] = jnp.sum(ex * oh2, axis=1, keepdims=True).astype(jnp.int32)
    carry[...] += jnp.sum(both, axis=0, keepdims=True)
    cnt_ref[...] = carry[...]


def _router(logits):
    return pl.pallas_call(
        _router_body,
        grid=(T // TMR,),
        in_specs=[
            pl.BlockSpec((TMR, E), lambda i: (i, 0)),
        ],
        out_specs=[
            pl.BlockSpec((TMR, 1), lambda i: (i, 0)),
            pl.BlockSpec((TMR, 1), lambda i: (i, 0)),
            pl.BlockSpec((TMR, 1), lambda i: (i, 0)),
            pl.BlockSpec((TMR, 1), lambda i: (i, 0)),
            pl.BlockSpec((TMR, 1), lambda i: (i, 0)),
            pl.BlockSpec((TMR, 1), lambda i: (i, 0)),
            pl.BlockSpec((1, E), lambda i: (0, 0)),
        ],
        out_shape=[
            jax.ShapeDtypeStruct((T, 1), jnp.int32),
            jax.ShapeDtypeStruct((T, 1), jnp.int32),
            jax.ShapeDtypeStruct((T, 1), jnp.float32),
            jax.ShapeDtypeStruct((T, 1), jnp.float32),
            jax.ShapeDtypeStruct((T, 1), jnp.int32),
            jax.ShapeDtypeStruct((T, 1), jnp.int32),
            jax.ShapeDtypeStruct((1, E), jnp.float32),
        ],
        scratch_shapes=[pltpu.VMEM((1, E), jnp.float32)],
    )(logits)


def _meta_body(cnt_ref, i1_ref, i2_ref, r0_ref, r1_ref,
               d0_ref, d1_ref, te_ref, act_ref):
    cnt = cnt_ref[...]
    padded = jnp.ceil(cnt * (1.0 / TM)) * TM
    er = lax.broadcasted_iota(jnp.int32, (E, E), 0)
    ec = lax.broadcasted_iota(jnp.int32, (E, E), 1)
    lt = (er < ec).astype(jnp.float32)
    starts = lax.dot_general(padded, lt, (((1,), (0,)), ((), ())),
                             precision=jax.lax.Precision.HIGHEST,
                             preferred_element_type=jnp.float32)
    ends = starts + padded
    iota1 = lax.broadcasted_iota(jnp.int32, (T, E), 1)
    s1 = jnp.sum(jnp.where(iota1 == i1_ref[...], starts, 0.0), axis=1,
                 keepdims=True)
    s2 = jnp.sum(jnp.where(iota1 == i2_ref[...], starts, 0.0), axis=1,
                 keepdims=True)
    d0_ref[...] = (s1 + r0_ref[...].astype(jnp.float32)).astype(jnp.int32)
    d1_ref[...] = (s2 + r1_ref[...].astype(jnp.float32)).astype(jnp.int32)

    tb = (lax.broadcasted_iota(jnp.int32, (NT, 1), 0) * TM).astype(jnp.float32)
    te_raw = jnp.minimum(
        jnp.sum((tb >= ends).astype(jnp.int32), axis=1, keepdims=True), E - 1)
    total = jnp.max(ends, axis=1, keepdims=True)
    act = (tb < total).astype(jnp.int32)
    nact = jnp.sum(act, axis=0, keepdims=True)
    ti = lax.broadcasted_iota(jnp.int32, (NT, 1), 0)
    last_e = jnp.sum(jnp.where(ti == nact - 1, te_raw, 0), axis=0,
                     keepdims=True)
    te_ref[...] = jnp.where(act == 1, te_raw, last_e)
    act_ref[...] = act


def _meta(cnt, i1, i2, r0, r1):
    return pl.pallas_call(
        _meta_body,
        out_shape=[
            jax.ShapeDtypeStruct((T, 1), jnp.int32),
            jax.ShapeDtypeStruct((T, 1), jnp.int32),
            jax.ShapeDtypeStruct((NT, 1), jnp.int32),
            jax.ShapeDtypeStruct((NT, 1), jnp.int32),
        ],
    )(cnt, i1, i2, r0, r1)


@functools.cache
def _sc_dispatch_fn():
    mesh = plsc.VectorSubcoreMesh(core_axis_name="c", subcore_axis_name="s")

    @functools.partial(
        pl.kernel,
        mesh=mesh,
        out_type=jax.ShapeDtypeStruct((P, H), jnp.float32),
        scratch_types=[
            pltpu.VMEM((2 * NCHD, CH), jnp.int32),
            pltpu.VMEM((CH, H), jnp.float32),
        ],
    )
    def _sc_dispatch(x_hbm, dst_hbm, xg_hbm, dst_v, rows_v):
        wid = lax.axis_index("s") * 2 + lax.axis_index("c")
        pltpu.sync_copy(dst_hbm.at[wid], dst_v)
        base = wid * TPW

        @pl.loop(0, NCHD)
        def _(c):
            pltpu.sync_copy(x_hbm.at[pl.ds(base + c * CH, CH)], rows_v)
            pltpu.sync_copy(rows_v, xg_hbm.at[dst_v.at[c]])
            pltpu.sync_copy(rows_v, xg_hbm.at[dst_v.at[NCHD + c]])

    return _sc_dispatch


@functools.cache
def _sc_gather2_fn():
    mesh = plsc.VectorSubcoreMesh(core_axis_name="c", subcore_axis_name="s")

    @functools.partial(
        pl.kernel,
        mesh=mesh,
        out_type=[jax.ShapeDtypeStruct((T, H), jnp.float32),
                  jax.ShapeDtypeStruct((T, H), jnp.float32)],
        scratch_types=[
            pltpu.VMEM((2, TPW), jnp.int32),
            pltpu.VMEM((CH, H), jnp.float32),
        ],
    )
    def _sc_gather2(y_hbm, idx0_hbm, idx1_hbm, y0_hbm, y1_hbm, idx_v, rows_v):
        wid = lax.axis_index("s") * 2 + lax.axis_index("c")
        base = wid * TPW
        pltpu.sync_copy(idx0_hbm.at[pl.ds(base, TPW)], idx_v.at[0])
        pltpu.sync_copy(idx1_hbm.at[pl.ds(base, TPW)], idx_v.at[1])

        @pl.loop(0, NCHD)
        def _(c):
            pltpu.sync_copy(y_hbm.at[idx_v.at[0].at[pl.ds(c * CH, CH)]], rows_v)
            pltpu.sync_copy(rows_v, y0_hbm.at[pl.ds(base + c * CH, CH)])
            pltpu.sync_copy(y_hbm.at[idx_v.at[1].at[pl.ds(c * CH, CH)]], rows_v)
            pltpu.sync_copy(rows_v, y1_hbm.at[pl.ds(base + c * CH, CH)])

    return _sc_gather2


def _mlp1_body(te_ref, act_ref, xg_ref, wg_ref, wu_ref, h_ref):
    @pl.when(act_ref[pl.program_id(1)] == 1)
    def _():
        x = xg_ref[...].astype(jnp.bfloat16)
        wg = wg_ref[0].astype(jnp.bfloat16)
        wu = wu_ref[0].astype(jnp.bfloat16)
        g = jax.lax.dot_general(x, wg, (((1,), (1,)), ((), ())),
                                preferred_element_type=jnp.float32)
        u = jax.lax.dot_general(x, wu, (((1,), (1,)), ((), ())),
                                preferred_element_type=jnp.float32)
        h_ref[...] = ((g * (1.0 / (1.0 + jnp.exp(-g)))) * u).astype(jnp.bfloat16)


def _mlp1(te, act, xg, Wg, Wu):
    grid_spec = pltpu.PrefetchScalarGridSpec(
        num_scalar_prefetch=2,
        grid=(NF, NT),
        in_specs=[
            pl.BlockSpec((TM, H), lambda j, i, te, act: (i, 0)),
            pl.BlockSpec((1, TF, H), lambda j, i, te, act: (te[i], j, 0)),
            pl.BlockSpec((1, TF, H), lambda j, i, te, act: (te[i], j, 0)),
        ],
        out_specs=pl.BlockSpec((TM, TF), lambda j, i, te, act: (i, j)),
    )
    return pl.pallas_call(
        _mlp1_body,
        grid_spec=grid_spec,
        out_shape=jax.ShapeDtypeStruct((P, FFN), jnp.bfloat16),
    )(te, act, xg, Wg, Wu)


def _mlp2_body(te_ref, act_ref, h_ref, wd_ref, bd_ref, y_ref, acc):
    @pl.when(act_ref[pl.program_id(0)] == 1)
    def _():
        k = pl.program_id(1)

        @pl.when(k == 0)
        def _():
            acc[...] = jnp.broadcast_to(bd_ref[0], acc.shape)

        h = h_ref[...]
        wd = wd_ref[0].astype(jnp.bfloat16)
        acc[...] += jax.lax.dot_general(h, wd, (((1,), (1,)), ((), ())),
                                        preferred_element_type=jnp.float32)

        @pl.when(k == NF - 1)
        def _():
            y_ref[...] = acc[...]


def _mlp2(te, act, h, Wd, bd):
    grid_spec = pltpu.PrefetchScalarGridSpec(
        num_scalar_prefetch=2,
        grid=(NT, NF),
        in_specs=[
            pl.BlockSpec((TM, TF), lambda i, k, te, act: (i, k)),
            pl.BlockSpec((1, H, TF), lambda i, k, te, act: (te[i], 0, k)),
            pl.BlockSpec((1, 1, H), lambda i, k, te, act: (te[i], 0, 0)),
        ],
        out_specs=pl.BlockSpec((TM, H), lambda i, k, te, act: (i, 0)),
        scratch_shapes=[pltpu.VMEM((TM, H), jnp.float32)],
    )
    return pl.pallas_call(
        _mlp2_body,
        grid_spec=grid_spec,
        out_shape=jax.ShapeDtypeStruct((P, H), jnp.float32),
    )(te, act, h, Wd, bd.reshape(E, 1, H))


def _combine_body(res_ref, y1_ref, y2_ref, w1_ref, w2_ref, out_ref):
    out_ref[...] = (res_ref[...]
                    + w1_ref[...] * y1_ref[...].astype(jnp.float32)
                    + w2_ref[...] * y2_ref[...].astype(jnp.float32))


def _combine(res, y1, y2, w1, w2):
    return pl.pallas_call(
        _combine_body,
        grid=(T // TMR,),
        in_specs=[
            pl.BlockSpec((TMR, H), lambda i: (i, 0)),
            pl.BlockSpec((TMR, H), lambda i: (i, 0)),
            pl.BlockSpec((TMR, H), lambda i: (i, 0)),
            pl.BlockSpec((TMR, 1), lambda i: (i, 0)),
            pl.BlockSpec((TMR, 1), lambda i: (i, 0)),
        ],
        out_specs=pl.BlockSpec((TMR, H), lambda i: (i, 0)),
        out_shape=jax.ShapeDtypeStruct((T, H), jnp.float32),
    )(res, y1, y2, w1, w2)


def kernel(hidden_states, residual, expert_limit, Wr, Wg, Wu, Wd, bd):
    b, s, h = hidden_states.shape
    x = hidden_states.reshape(T, H)

    router_logits = jnp.einsum("bsh,eh->bse", hidden_states, Wr)
    expert_mask = jnp.arange(E) < expert_limit
    router_logits = jnp.where(expert_mask, router_logits, -jnp.inf)
    logits = router_logits.reshape(-1, E)
    logits_f = jnp.where(jnp.isfinite(logits), logits, NEG)
    i1, i2, w1, w2, r0, r1, cnt = _router(logits_f)
    d0, d1, te2, act2 = _meta(cnt, i1, i2, r0, r1)
    te = te2.reshape(NT)
    act = act2.reshape(NT)

    dst_sw = jnp.concatenate(
        [d0.reshape(NW, NCHD, CH), d1.reshape(NW, NCHD, CH)], axis=1)
    xg = _sc_dispatch_fn()(x, dst_sw)

    yg = _mlp2(te, act, _mlp1(te, act, xg, Wg, Wu), Wd, bd)

    y1, y2 = _sc_gather2_fn()(yg, d0.reshape(T), d1.reshape(T))

    out = _combine(residual.reshape(T, H), y1, y2, w1, w2)
    return out.reshape(b, s, h), logits


# speedup vs baseline: 2.6283x; 1.0074x over previous
import functools

import jax
import jax.numpy as jnp
from jax import lax
from jax.experimental import pallas as pl
from jax.experimental.pallas import tpu as pltpu
from jax.experimental.pallas import tpu_sc as plsc

E = 8
H = 1024
FFN = 2816
T = 4096
A = 2 * T
TM = 512
P = A + E * TM
NT = P // TM
TF = 1408
NF = FFN // TF
NEG = -1e30

NW = 32
TPW = T // NW
CH = 64
NCHD = TPW // CH

TMR = 512


def _router_body(lg_ref, i1_ref, i2_ref, w1_ref, w2_ref, r0_ref, r1_ref,
                 cnt_ref, carry):
    step = pl.program_id(0)

    @pl.when(step == 0)
    def _():
        carry[...] = jnp.zeros_like(carry)

    logits = lg_ref[...]
    iota = lax.broadcasted_iota(jnp.int32, logits.shape, 1)
    l1 = jnp.max(logits, axis=1, keepdims=True)
    i1 = jnp.min(jnp.where(logits == l1, iota, E), axis=1, keepdims=True)
    m2 = jnp.where(iota == i1, NEG, logits)
    l2 = jnp.max(m2, axis=1, keepdims=True)
    i2 = jnp.min(jnp.where(m2 == l2, iota, E), axis=1, keepdims=True)
    w1 = 1.0 / (1.0 + jnp.exp(l2 - l1))
    i1_ref[...] = i1
    i2_ref[...] = i2
    w1_ref[...] = w1
    w2_ref[...] = 1.0 - w1

    oh1 = (iota == i1).astype(jnp.float32)
    oh2 = (iota == i2).astype(jnp.float32)
    both = oh1 + oh2
    rr = lax.broadcasted_iota(jnp.int32, (TMR, TMR), 0)
    cc = lax.broadcasted_iota(jnp.int32, (TMR, TMR), 1)
    tri = (cc < rr).astype(jnp.float32)
    ex = lax.dot_general(tri, both, (((1,), (0,)), ((), ())),
                         precision=jax.lax.Precision.HIGHEST,
                         preferred_element_type=jnp.float32) + carry[...]
    r0_ref[...] = jnp.sum(ex * oh1, axis=1, keepdims=True).astype(jnp.int32)
    r1_ref[...] = jnp.sum(ex * oh2, axis=1, keepdims=True).astype(jnp.int32)
    carry[...] += jnp.sum(both, axis=0, keepdims=True)
    cnt_ref[...] = carry[...]


def _router(logits):
    return pl.pallas_call(
        _router_body,
        grid=(T // TMR,),
        in_specs=[
            pl.BlockSpec((TMR, E), lambda i: (i, 0)),
        ],
        out_specs=[
            pl.BlockSpec((TMR, 1), lambda i: (i, 0)),
            pl.BlockSpec((TMR, 1), lambda i: (i, 0)),
            pl.BlockSpec((TMR, 1), lambda i: (i, 0)),
            pl.BlockSpec((TMR, 1), lambda i: (i, 0)),
            pl.BlockSpec((TMR, 1), lambda i: (i, 0)),
            pl.BlockSpec((TMR, 1), lambda i: (i, 0)),
            pl.BlockSpec((1, E), lambda i: (0, 0)),
        ],
        out_shape=[
            jax.ShapeDtypeStruct((T, 1), jnp.int32),
            jax.ShapeDtypeStruct((T, 1), jnp.int32),
            jax.ShapeDtypeStruct((T, 1), jnp.float32),
            jax.ShapeDtypeStruct((T, 1), jnp.float32),
            jax.ShapeDtypeStruct((T, 1), jnp.int32),
            jax.ShapeDtypeStruct((T, 1), jnp.int32),
            jax.ShapeDtypeStruct((1, E), jnp.float32),
        ],
        scratch_shapes=[pltpu.VMEM((1, E), jnp.float32)],
    )(logits)


def _meta_body(cnt_ref, i1_ref, i2_ref, r0_ref, r1_ref,
               d0_ref, d1_ref, te_ref, act_ref):
    cnt = cnt_ref[...]
    padded = jnp.ceil(cnt * (1.0 / TM)) * TM
    er = lax.broadcasted_iota(jnp.int32, (E, E), 0)
    ec = lax.broadcasted_iota(jnp.int32, (E, E), 1)
    lt = (er < ec).astype(jnp.float32)
    starts = lax.dot_general(padded, lt, (((1,), (0,)), ((), ())),
                             precision=jax.lax.Precision.HIGHEST,
                             preferred_element_type=jnp.float32)
    ends = starts + padded
    iota1 = lax.broadcasted_iota(jnp.int32, (T, E), 1)
    s1 = jnp.sum(jnp.where(iota1 == i1_ref[...], starts, 0.0), axis=1,
                 keepdims=True)
    s2 = jnp.sum(jnp.where(iota1 == i2_ref[...], starts, 0.0), axis=1,
                 keepdims=True)
    d0_ref[...] = (s1 + r0_ref[...].astype(jnp.float32)).astype(jnp.int32)
    d1_ref[...] = (s2 + r1_ref[...].astype(jnp.float32)).astype(jnp.int32)

    tb = (lax.broadcasted_iota(jnp.int32, (NT, 1), 0) * TM).astype(jnp.float32)
    te_raw = jnp.minimum(
        jnp.sum((tb >= ends).astype(jnp.int32), axis=1, keepdims=True), E - 1)
    total = jnp.max(ends, axis=1, keepdims=True)
    act = (tb < total).astype(jnp.int32)
    nact = jnp.sum(act, axis=0, keepdims=True)
    ti = lax.broadcasted_iota(jnp.int32, (NT, 1), 0)
    last_e = jnp.sum(jnp.where(ti == nact - 1, te_raw, 0), axis=0,
                     keepdims=True)
    te_ref[...] = jnp.where(act == 1, te_raw, last_e)
    act_ref[...] = act


def _meta(cnt, i1, i2, r0, r1):
    return pl.pallas_call(
        _meta_body,
        out_shape=[
            jax.ShapeDtypeStruct((T, 1), jnp.int32),
            jax.ShapeDtypeStruct((T, 1), jnp.int32),
            jax.ShapeDtypeStruct((NT, 1), jnp.int32),
            jax.ShapeDtypeStruct((NT, 1), jnp.int32),
        ],
    )(cnt, i1, i2, r0, r1)


@functools.cache
def _sc_dispatch_fn():
    mesh = plsc.VectorSubcoreMesh(core_axis_name="c", subcore_axis_name="s")

    @functools.partial(
        pl.kernel,
        mesh=mesh,
        out_type=jax.ShapeDtypeStruct((P, H), jnp.float32),
        scratch_types=[
            pltpu.VMEM((2 * NCHD, CH), jnp.int32),
            pltpu.VMEM((CH, H), jnp.float32),
        ],
    )
    def _sc_dispatch(x_hbm, dst_hbm, xg_hbm, dst_v, rows_v):
        wid = lax.axis_index("s") * 2 + lax.axis_index("c")
        pltpu.sync_copy(dst_hbm.at[wid], dst_v)
        base = wid * TPW

        @pl.loop(0, NCHD)
        def _(c):
            pltpu.sync_copy(x_hbm.at[pl.ds(base + c * CH, CH)], rows_v)
            pltpu.sync_copy(rows_v, xg_hbm.at[dst_v.at[c]])
            pltpu.sync_copy(rows_v, xg_hbm.at[dst_v.at[NCHD + c]])

    return _sc_dispatch


@functools.cache
def _sc_gather2_fn():
    mesh = plsc.VectorSubcoreMesh(core_axis_name="c", subcore_axis_name="s")

    @functools.partial(
        pl.kernel,
        mesh=mesh,
        out_type=[jax.ShapeDtypeStruct((T, H), jnp.float32),
                  jax.ShapeDtypeStruct((T, H), jnp.float32)],
        scratch_types=[
            pltpu.VMEM((2, TPW), jnp.int32),
            pltpu.VMEM((CH, H), jnp.float32),
        ],
    )
    def _sc_gather2(y_hbm, idx0_hbm, idx1_hbm, y0_hbm, y1_hbm, idx_v, rows_v):
        wid = lax.axis_index("s") * 2 + lax.axis_index("c")
        base = wid * TPW
        pltpu.sync_copy(idx0_hbm.at[pl.ds(base, TPW)], idx_v.at[0])
        pltpu.sync_copy(idx1_hbm.at[pl.ds(base, TPW)], idx_v.at[1])

        @pl.loop(0, NCHD)
        def _(c):
            pltpu.sync_copy(y_hbm.at[idx_v.at[0].at[pl.ds(c * CH, CH)]], rows_v)
            pltpu.sync_copy(rows_v, y0_hbm.at[pl.ds(base + c * CH, CH)])
            pltpu.sync_copy(y_hbm.at[idx_v.at[1].at[pl.ds(c * CH, CH)]], rows_v)
            pltpu.sync_copy(rows_v, y1_hbm.at[pl.ds(base + c * CH, CH)])

    return _sc_gather2


def _mlp1_body(te_ref, act_ref, xg_ref, wg_ref, wu_ref, h_ref):
    @pl.when(act_ref[pl.program_id(0) // 2] == 1)
    def _():
        x = xg_ref[...].astype(jnp.bfloat16)
        wg = wg_ref[0].astype(jnp.bfloat16)
        wu = wu_ref[0].astype(jnp.bfloat16)
        g = jax.lax.dot_general(x, wg, (((1,), (1,)), ((), ())),
                                preferred_element_type=jnp.float32)
        u = jax.lax.dot_general(x, wu, (((1,), (1,)), ((), ())),
                                preferred_element_type=jnp.float32)
        h_ref[...] = ((g * (1.0 / (1.0 + jnp.exp(-g)))) * u).astype(jnp.bfloat16)


def _mlp1(te, act, xg, Wg, Wu):
    TM1 = TM // 2
    grid_spec = pltpu.PrefetchScalarGridSpec(
        num_scalar_prefetch=2,
        grid=(2 * NT,),
        in_specs=[
            pl.BlockSpec((TM1, H), lambda i, te, act: (i, 0)),
            pl.BlockSpec((1, FFN, H), lambda i, te, act: (te[i // 2], 0, 0)),
            pl.BlockSpec((1, FFN, H), lambda i, te, act: (te[i // 2], 0, 0)),
        ],
        out_specs=pl.BlockSpec((TM1, FFN), lambda i, te, act: (i, 0)),
    )
    return pl.pallas_call(
        _mlp1_body,
        grid_spec=grid_spec,
        out_shape=jax.ShapeDtypeStruct((P, FFN), jnp.bfloat16),
    )(te, act, xg, Wg, Wu)


def _mlp2_body(te_ref, act_ref, h_ref, wd_ref, bd_ref, y_ref, acc):
    @pl.when(act_ref[pl.program_id(0)] == 1)
    def _():
        k = pl.program_id(1)

        @pl.when(k == 0)
        def _():
            acc[...] = jnp.broadcast_to(bd_ref[0], acc.shape)

        h = h_ref[...]
        wd = wd_ref[0].astype(jnp.bfloat16)
        acc[...] += jax.lax.dot_general(h, wd, (((1,), (1,)), ((), ())),
                                        preferred_element_type=jnp.float32)

        @pl.when(k == NF - 1)
        def _():
            y_ref[...] = acc[...]


def _mlp2(te, act, h, Wd, bd):
    grid_spec = pltpu.PrefetchScalarGridSpec(
        num_scalar_prefetch=2,
        grid=(NT, NF),
        in_specs=[
            pl.BlockSpec((TM, TF), lambda i, k, te, act: (i, k)),
            pl.BlockSpec((1, H, TF), lambda i, k, te, act: (te[i], 0, k)),
            pl.BlockSpec((1, 1, H), lambda i, k, te, act: (te[i], 0, 0)),
        ],
        out_specs=pl.BlockSpec((TM, H), lambda i, k, te, act: (i, 0)),
        scratch_shapes=[pltpu.VMEM((TM, H), jnp.float32)],
    )
    return pl.pallas_call(
        _mlp2_body,
        grid_spec=grid_spec,
        out_shape=jax.ShapeDtypeStruct((P, H), jnp.float32),
    )(te, act, h, Wd, bd.reshape(E, 1, H))


def _combine_body(res_ref, y1_ref, y2_ref, w1_ref, w2_ref, out_ref):
    out_ref[...] = (res_ref[...]
                    + w1_ref[...] * y1_ref[...].astype(jnp.float32)
                    + w2_ref[...] * y2_ref[...].astype(jnp.float32))


def _combine(res, y1, y2, w1, w2):
    return pl.pallas_call(
        _combine_body,
        grid=(T // TMR,),
        in_specs=[
            pl.BlockSpec((TMR, H), lambda i: (i, 0)),
            pl.BlockSpec((TMR, H), lambda i: (i, 0)),
            pl.BlockSpec((TMR, H), lambda i: (i, 0)),
            pl.BlockSpec((TMR, 1), lambda i: (i, 0)),
            pl.BlockSpec((TMR, 1), lambda i: (i, 0)),
        ],
        out_specs=pl.BlockSpec((TMR, H), lambda i: (i, 0)),
        out_shape=jax.ShapeDtypeStruct((T, H), jnp.float32),
    )(res, y1, y2, w1, w2)


def kernel(hidden_states, residual, expert_limit, Wr, Wg, Wu, Wd, bd):
    b, s, h = hidden_states.shape
    x = hidden_states.reshape(T, H)

    router_logits = jnp.einsum("bsh,eh->bse", hidden_states, Wr)
    expert_mask = jnp.arange(E) < expert_limit
    router_logits = jnp.where(expert_mask, router_logits, -jnp.inf)
    logits = router_logits.reshape(-1, E)
    logits_f = jnp.where(jnp.isfinite(logits), logits, NEG)
    i1, i2, w1, w2, r0, r1, cnt = _router(logits_f)
    d0, d1, te2, act2 = _meta(cnt, i1, i2, r0, r1)
    te = te2.reshape(NT)
    act = act2.reshape(NT)

    dst_sw = jnp.concatenate(
        [d0.reshape(NW, NCHD, CH), d1.reshape(NW, NCHD, CH)], axis=1)
    xg = _sc_dispatch_fn()(x, dst_sw)

    yg = _mlp2(te, act, _mlp1(te, act, xg, Wg, Wu), Wd, bd)

    y1, y2 = _sc_gather2_fn()(yg, d0.reshape(T), d1.reshape(T))

    out = _combine(residual.reshape(T, H), y1, y2, w1, w2)
    return out.reshape(b, s, h), logits


# speedup vs baseline: 2.8583x; 1.0875x over previous
import functools

import jax
import jax.numpy as jnp
from jax import lax
from jax.experimental import pallas as pl
from jax.experimental.pallas import tpu as pltpu
from jax.experimental.pallas import tpu_sc as plsc

E = 8
H = 1024
FFN = 2816
T = 4096
A = 2 * T
TM = 512
P = A + E * TM
NT = P // TM
TF = 1408
NF = FFN // TF
NEG = -1e30

NW = 32
TPW = T // NW
CH = 64
NCHD = TPW // CH

TMR = 512


def _router_body(lg_ref, i1_ref, i2_ref, w1_ref, w2_ref, r0_ref, r1_ref,
                 cnt_ref, carry):
    step = pl.program_id(0)

    @pl.when(step == 0)
    def _():
        carry[...] = jnp.zeros_like(carry)

    logits = lg_ref[...]
    iota = lax.broadcasted_iota(jnp.int32, logits.shape, 1)
    l1 = jnp.max(logits, axis=1, keepdims=True)
    i1 = jnp.min(jnp.where(logits == l1, iota, E), axis=1, keepdims=True)
    m2 = jnp.where(iota == i1, NEG, logits)
    l2 = jnp.max(m2, axis=1, keepdims=True)
    i2 = jnp.min(jnp.where(m2 == l2, iota, E), axis=1, keepdims=True)
    w1 = 1.0 / (1.0 + jnp.exp(l2 - l1))
    i1_ref[...] = i1
    i2_ref[...] = i2
    w1_ref[...] = w1
    w2_ref[...] = 1.0 - w1

    oh1 = (iota == i1).astype(jnp.float32)
    oh2 = (iota == i2).astype(jnp.float32)
    both = oh1 + oh2
    rr = lax.broadcasted_iota(jnp.int32, (TMR, TMR), 0)
    cc = lax.broadcasted_iota(jnp.int32, (TMR, TMR), 1)
    tri = (cc < rr).astype(jnp.float32)
    ex = lax.dot_general(tri, both, (((1,), (0,)), ((), ())),
                         precision=jax.lax.Precision.HIGHEST,
                         preferred_element_type=jnp.float32) + carry[...]
    r0_ref[...] = jnp.sum(ex * oh1, axis=1, keepdims=True).astype(jnp.int32)
    r1_ref[...] = jnp.sum(ex * oh2, axis=1, keepdims=True).astype(jnp.int32)
    carry[...] += jnp.sum(both, axis=0, keepdims=True)
    cnt_ref[...] = carry[...]


def _router(logits):
    return pl.pallas_call(
        _router_body,
        grid=(T // TMR,),
        in_specs=[
            pl.BlockSpec((TMR, E), lambda i: (i, 0)),
        ],
        out_specs=[
            pl.BlockSpec((TMR, 1), lambda i: (i, 0)),
            pl.BlockSpec((TMR, 1), lambda i: (i, 0)),
            pl.BlockSpec((TMR, 1), lambda i: (i, 0)),
            pl.BlockSpec((TMR, 1), lambda i: (i, 0)),
            pl.BlockSpec((TMR, 1), lambda i: (i, 0)),
            pl.BlockSpec((TMR, 1), lambda i: (i, 0)),
            pl.BlockSpec((1, E), lambda i: (0, 0)),
        ],
        out_shape=[
            jax.ShapeDtypeStruct((T, 1), jnp.int32),
            jax.ShapeDtypeStruct((T, 1), jnp.int32),
            jax.ShapeDtypeStruct((T, 1), jnp.float32),
            jax.ShapeDtypeStruct((T, 1), jnp.float32),
            jax.ShapeDtypeStruct((T, 1), jnp.int32),
            jax.ShapeDtypeStruct((T, 1), jnp.int32),
            jax.ShapeDtypeStruct((1, E), jnp.float32),
        ],
        scratch_shapes=[pltpu.VMEM((1, E), jnp.float32)],
    )(logits)


def _meta_body(cnt_ref, i1_ref, i2_ref, r0_ref, r1_ref,
               d0_ref, d1_ref, te_ref, act_ref):
    cnt = cnt_ref[...]
    padded = jnp.ceil(cnt * (1.0 / TM)) * TM
    er = lax.broadcasted_iota(jnp.int32, (E, E), 0)
    ec = lax.broadcasted_iota(jnp.int32, (E, E), 1)
    lt = (er < ec).astype(jnp.float32)
    starts = lax.dot_general(padded, lt, (((1,), (0,)), ((), ())),
                             precision=jax.lax.Precision.HIGHEST,
                             preferred_element_type=jnp.float32)
    ends = starts + padded
    iota1 = lax.broadcasted_iota(jnp.int32, (T, E), 1)
    s1 = jnp.sum(jnp.where(iota1 == i1_ref[...], starts, 0.0), axis=1,
                 keepdims=True)
    s2 = jnp.sum(jnp.where(iota1 == i2_ref[...], starts, 0.0), axis=1,
                 keepdims=True)
    d0_ref[...] = (s1 + r0_ref[...].astype(jnp.float32)).astype(jnp.int32)
    d1_ref[...] = (s2 + r1_ref[...].astype(jnp.float32)).astype(jnp.int32)

    tb = (lax.broadcasted_iota(jnp.int32, (NT, 1), 0) * TM).astype(jnp.float32)
    te_raw = jnp.minimum(
        jnp.sum((tb >= ends).astype(jnp.int32), axis=1, keepdims=True), E - 1)
    total = jnp.max(ends, axis=1, keepdims=True)
    act = (tb < total).astype(jnp.int32)
    nact = jnp.sum(act, axis=0, keepdims=True)
    ti = lax.broadcasted_iota(jnp.int32, (NT, 1), 0)
    last_e = jnp.sum(jnp.where(ti == nact - 1, te_raw, 0), axis=0,
                     keepdims=True)
    te_ref[...] = jnp.where(act == 1, te_raw, last_e)
    act_ref[...] = act


def _meta(cnt, i1, i2, r0, r1):
    return pl.pallas_call(
        _meta_body,
        out_shape=[
            jax.ShapeDtypeStruct((T, 1), jnp.int32),
            jax.ShapeDtypeStruct((T, 1), jnp.int32),
            jax.ShapeDtypeStruct((NT, 1), jnp.int32),
            jax.ShapeDtypeStruct((NT, 1), jnp.int32),
        ],
    )(cnt, i1, i2, r0, r1)


@functools.cache
def _sc_dispatch_fn():
    mesh = plsc.VectorSubcoreMesh(core_axis_name="c", subcore_axis_name="s")

    @functools.partial(
        pl.kernel,
        mesh=mesh,
        out_type=jax.ShapeDtypeStruct((P, H), jnp.float32),
        scratch_types=[
            pltpu.VMEM((2 * NCHD, CH), jnp.int32),
            pltpu.VMEM((CH, H), jnp.float32),
        ],
    )
    def _sc_dispatch(x_hbm, dst_hbm, xg_hbm, dst_v, rows_v):
        wid = lax.axis_index("s") * 2 + lax.axis_index("c")
        pltpu.sync_copy(dst_hbm.at[wid], dst_v)
        base = wid * TPW

        @pl.loop(0, NCHD)
        def _(c):
            pltpu.sync_copy(x_hbm.at[pl.ds(base + c * CH, CH)], rows_v)
            pltpu.sync_copy(rows_v, xg_hbm.at[dst_v.at[c]])
            pltpu.sync_copy(rows_v, xg_hbm.at[dst_v.at[NCHD + c]])

    return _sc_dispatch


@functools.cache
def _sc_gather2_fn():
    mesh = plsc.VectorSubcoreMesh(core_axis_name="c", subcore_axis_name="s")

    @functools.partial(
        pl.kernel,
        mesh=mesh,
        out_type=[jax.ShapeDtypeStruct((T, H), jnp.float32),
                  jax.ShapeDtypeStruct((T, H), jnp.float32)],
        scratch_types=[
            pltpu.VMEM((2, TPW), jnp.int32),
            pltpu.VMEM((CH, H), jnp.float32),
        ],
    )
    def _sc_gather2(y_hbm, idx0_hbm, idx1_hbm, y0_hbm, y1_hbm, idx_v, rows_v):
        wid = lax.axis_index("s") * 2 + lax.axis_index("c")
        base = wid * TPW
        pltpu.sync_copy(idx0_hbm.at[pl.ds(base, TPW)], idx_v.at[0])
        pltpu.sync_copy(idx1_hbm.at[pl.ds(base, TPW)], idx_v.at[1])

        @pl.loop(0, NCHD)
        def _(c):
            pltpu.sync_copy(y_hbm.at[idx_v.at[0].at[pl.ds(c * CH, CH)]], rows_v)
            pltpu.sync_copy(rows_v, y0_hbm.at[pl.ds(base + c * CH, CH)])
            pltpu.sync_copy(y_hbm.at[idx_v.at[1].at[pl.ds(c * CH, CH)]], rows_v)
            pltpu.sync_copy(rows_v, y1_hbm.at[pl.ds(base + c * CH, CH)])

    return _sc_gather2


def _mlp1_body(te_ref, act_ref, xg_ref, wg_ref, wu_ref, h_ref):
    @pl.when(act_ref[pl.program_id(0) // 2] == 1)
    def _():
        x = xg_ref[...].astype(jnp.bfloat16)
        wg = wg_ref[0].astype(jnp.bfloat16)
        wu = wu_ref[0].astype(jnp.bfloat16)
        g = jax.lax.dot_general(x, wg, (((1,), (1,)), ((), ())),
                                preferred_element_type=jnp.float32)
        u = jax.lax.dot_general(x, wu, (((1,), (1,)), ((), ())),
                                preferred_element_type=jnp.float32)
        h_ref[...] = ((g * (1.0 / (1.0 + jnp.exp(-g)))) * u).astype(jnp.bfloat16)


def _mlp1(te, act, xg, Wg, Wu):
    TM1 = TM // 2
    grid_spec = pltpu.PrefetchScalarGridSpec(
        num_scalar_prefetch=2,
        grid=(2 * NT,),
        in_specs=[
            pl.BlockSpec((TM1, H), lambda i, te, act: (i, 0)),
            pl.BlockSpec((1, FFN, H), lambda i, te, act: (te[i // 2], 0, 0)),
            pl.BlockSpec((1, FFN, H), lambda i, te, act: (te[i // 2], 0, 0)),
        ],
        out_specs=pl.BlockSpec((TM1, FFN), lambda i, te, act: (i, 0)),
    )
    return pl.pallas_call(
        _mlp1_body,
        grid_spec=grid_spec,
        out_shape=jax.ShapeDtypeStruct((P, FFN), jnp.bfloat16),
    )(te, act, xg, Wg, Wu)


def _mlp2_body(te_ref, act_ref, h_ref, wd_ref, bd_ref, y_ref):
    @pl.when(act_ref[pl.program_id(0)] == 1)
    def _():
        h = h_ref[...]
        wd = wd_ref[0].astype(jnp.bfloat16)
        y_ref[...] = jnp.broadcast_to(bd_ref[0], y_ref.shape) + \
            jax.lax.dot_general(h, wd, (((1,), (1,)), ((), ())),
                                preferred_element_type=jnp.float32)


def _mlp2(te, act, h, Wd, bd):
    grid_spec = pltpu.PrefetchScalarGridSpec(
        num_scalar_prefetch=2,
        grid=(NT,),
        in_specs=[
            pl.BlockSpec((TM, FFN), lambda i, te, act: (i, 0)),
            pl.BlockSpec((1, H, FFN), lambda i, te, act: (te[i], 0, 0)),
            pl.BlockSpec((1, 1, H), lambda i, te, act: (te[i], 0, 0)),
        ],
        out_specs=pl.BlockSpec((TM, H), lambda i, te, act: (i, 0)),
    )
    return pl.pallas_call(
        _mlp2_body,
        grid_spec=grid_spec,
        out_shape=jax.ShapeDtypeStruct((P, H), jnp.float32),
    )(te, act, h, Wd, bd.reshape(E, 1, H))


def _combine_body(res_ref, y1_ref, y2_ref, w1_ref, w2_ref, out_ref):
    out_ref[...] = (res_ref[...]
                    + w1_ref[...] * y1_ref[...].astype(jnp.float32)
                    + w2_ref[...] * y2_ref[...].astype(jnp.float32))


def _combine(res, y1, y2, w1, w2):
    return pl.pallas_call(
        _combine_body,
        grid=(T // TMR,),
        in_specs=[
            pl.BlockSpec((TMR, H), lambda i: (i, 0)),
            pl.BlockSpec((TMR, H), lambda i: (i, 0)),
            pl.BlockSpec((TMR, H), lambda i: (i, 0)),
            pl.BlockSpec((TMR, 1), lambda i: (i, 0)),
            pl.BlockSpec((TMR, 1), lambda i: (i, 0)),
        ],
        out_specs=pl.BlockSpec((TMR, H), lambda i: (i, 0)),
        out_shape=jax.ShapeDtypeStruct((T, H), jnp.float32),
    )(res, y1, y2, w1, w2)


def kernel(hidden_states, residual, expert_limit, Wr, Wg, Wu, Wd, bd):
    b, s, h = hidden_states.shape
    x = hidden_states.reshape(T, H)

    router_logits = jnp.einsum("bsh,eh->bse", hidden_states, Wr)
    expert_mask = jnp.arange(E) < expert_limit
    router_logits = jnp.where(expert_mask, router_logits, -jnp.inf)
    logits = router_logits.reshape(-1, E)
    logits_f = jnp.where(jnp.isfinite(logits), logits, NEG)
    i1, i2, w1, w2, r0, r1, cnt = _router(logits_f)
    d0, d1, te2, act2 = _meta(cnt, i1, i2, r0, r1)
    te = te2.reshape(NT)
    act = act2.reshape(NT)

    dst_sw = jnp.concatenate(
        [d0.reshape(NW, NCHD, CH), d1.reshape(NW, NCHD, CH)], axis=1)
    xg = _sc_dispatch_fn()(x, dst_sw)

    yg = _mlp2(te, act, _mlp1(te, act, xg, Wg, Wu), Wd, bd)

    y1, y2 = _sc_gather2_fn()(yg, d0.reshape(T), d1.reshape(T))

    out = _combine(residual.reshape(T, H), y1, y2, w1, w2)
    return out.reshape(b, s, h), logits
